# baseline (speedup 1.0000x reference)
import jax
import jax.numpy as jnp
from jax import lax
from jax.experimental import pallas as pl
from jax.experimental.pallas import tpu as pltpu
from jax.experimental.pallas import tpu_sc as plsc

_NC = 2
_NS = 16
_NW = _NC * _NS
_CHUNK = 128
_IDX_BLK = 32
_BL = 2048


def _mm1_body(xt_ref, wat_ref, ba_ref, wbt_ref, bb_ref, ht_ref, ft_ref):
    ht = jnp.dot(wat_ref[...], xt_ref[...], preferred_element_type=jnp.float32)
    ht = ht + ba_ref[...]
    ht_ref[...] = ht
    ft_ref[...] = (
        jnp.dot(wbt_ref[...], ht, preferred_element_type=jnp.float32)
        + bb_ref[...]
    )


def _layer1_dense(xt, wat, ba, wbt, bb):
    f_in, n = xt.shape
    h_dim = wat.shape[0]
    grid = (-(-n // _BL),)
    n_p = grid[0] * _BL
    return pl.pallas_call(
        _mm1_body,
        grid=grid,
        in_specs=[
            pl.BlockSpec((f_in, _BL), lambda i: (0, i)),
            pl.BlockSpec((h_dim, f_in), lambda i: (0, 0)),
            pl.BlockSpec((h_dim, 1), lambda i: (0, 0)),
            pl.BlockSpec((h_dim, h_dim), lambda i: (0, 0)),
            pl.BlockSpec((h_dim, 1), lambda i: (0, 0)),
        ],
        out_specs=[
            pl.BlockSpec((h_dim, _BL), lambda i: (0, i)),
            pl.BlockSpec((h_dim, _BL), lambda i: (0, i)),
        ],
        out_shape=[
            jax.ShapeDtypeStruct((h_dim, n_p), jnp.float32),
            jax.ShapeDtypeStruct((h_dim, n_p), jnp.float32),
        ],
    )(xt, wat, ba.reshape(-1, 1), wbt, bb.reshape(-1, 1))


def _make_edge_scatter(n_rows_pad, n_chunks, per_core_src):
    rows_per_tile = n_rows_pad // _NS
    n_outer = n_chunks // _IDX_BLK
    mesh = plsc.VectorSubcoreMesh(core_axis_name="c", subcore_axis_name="s")

    def body(table_hbm, src_hbm, dst_hbm, z_hbm, out_hbm,
             srcb, dstb, grow, acc, gsem, ssem):
        c = lax.axis_index("c")
        s = lax.axis_index("s")

        pltpu.sync_copy(z_hbm, acc.at[pl.ds(s * rows_per_tile, rows_per_tile)])
        plsc.subcore_barrier()

        def outer(ob, carry):
            base = ob * _IDX_BLK
            if per_core_src:
                pltpu.sync_copy(src_hbm.at[c, s, pl.ds(base, _IDX_BLK)], srcb)
                pltpu.sync_copy(dst_hbm.at[s, pl.ds(base, _IDX_BLK)], dstb)
            else:
                wid = s * _NC + c
                pltpu.sync_copy(src_hbm.at[wid, pl.ds(base, _IDX_BLK)], srcb)
                pltpu.sync_copy(dst_hbm.at[wid, pl.ds(base, _IDX_BLK)], dstb)
            gathers = [
                pltpu.async_copy(table_hbm.at[srcb.at[j]], grow.at[j], gsem)
                for j in range(_IDX_BLK)
            ]
            for g in gathers:
                g.wait()
            scats = [
                pltpu.async_copy(grow.at[j], acc.at[dstb.at[j]], ssem, add=True)
                for j in range(_IDX_BLK)
            ]
            for sc in scats:
                sc.wait()
            return carry

        lax.fori_loop(0, n_outer, outer, 0)
        plsc.subcore_barrier()
        pltpu.sync_copy(
            acc.at[pl.ds(s * rows_per_tile, rows_per_tile)],
            out_hbm.at[c, pl.ds(s * rows_per_tile, rows_per_tile)],
        )

    return pl.kernel(
        body,
        out_type=jax.ShapeDtypeStruct((_NC, n_rows_pad, 8), jnp.float32),
        mesh=mesh,
        compiler_params=pltpu.CompilerParams(use_tc_tiling_on_sc=False),
        scratch_types=[
            pltpu.VMEM((_IDX_BLK, _CHUNK), jnp.int32),
            pltpu.VMEM((_IDX_BLK, _CHUNK), jnp.int32),
            pltpu.VMEM((_IDX_BLK, _CHUNK, 8), jnp.float32),
            pltpu.VMEM_SHARED((n_rows_pad, 8), jnp.float32),
            pltpu.SemaphoreType.DMA,
            pltpu.SemaphoreType.DMA,
        ],
    )


def _mid_body(f_ref, p_ref, wat_ref, ba_ref, wbt_ref, bb_ref, h2_ref, f2_ref):
    agg = jnp.concatenate([p_ref[0], p_ref[1]], axis=0)
    out1 = jnp.maximum(f_ref[...] + agg, 0.0)
    h2 = jnp.dot(wat_ref[...], out1, preferred_element_type=jnp.float32)
    h2 = h2 + ba_ref[...]
    h2_ref[...] = h2
    f2_ref[...] = (
        jnp.dot(wbt_ref[...], h2, preferred_element_type=jnp.float32)
        + bb_ref[...]
    )


def _mid_dense(f1t, aggt, wat_p, ba_p, wbt_p, bb_p):
    h_dim, n_p = f1t.shape
    grid = (n_p // _BL,)
    return pl.pallas_call(
        _mid_body,
        grid=grid,
        in_specs=[
            pl.BlockSpec((h_dim, _BL), lambda i: (0, i)),
            pl.BlockSpec((_NC, 8, _BL), lambda i: (0, 0, i)),
            pl.BlockSpec((8, h_dim), lambda i: (0, 0)),
            pl.BlockSpec((8, 1), lambda i: (0, 0)),
            pl.BlockSpec((8, 8), lambda i: (0, 0)),
            pl.BlockSpec((8, 1), lambda i: (0, 0)),
        ],
        out_specs=[
            pl.BlockSpec((8, _BL), lambda i: (0, i)),
            pl.BlockSpec((8, _BL), lambda i: (0, i)),
        ],
        out_shape=[
            jax.ShapeDtypeStruct((8, n_p), jnp.float32),
            jax.ShapeDtypeStruct((8, n_p), jnp.float32),
        ],
    )(f1t, aggt, wat_p, ba_p, wbt_p, bb_p)


def _make_final_body(n_cls):
    def _final_body(f2_ref, p_ref, o_ref):
        z = f2_ref[...] + p_ref[0] + p_ref[1]
        row = lax.broadcasted_iota(jnp.int32, z.shape, 0)
        z = jnp.where(row < n_cls, z, -jnp.inf)
        m = jnp.max(z, axis=0, keepdims=True)
        zs = z - m
        lse = jnp.log(jnp.sum(jnp.exp(zs), axis=0, keepdims=True))
        o_ref[...] = (zs - lse)[:n_cls, :]
    return _final_body


def _final_dense(f2t, aggt2, n_cls):
    _, n_p = f2t.shape
    grid = (n_p // _BL,)
    return pl.pallas_call(
        _make_final_body(n_cls),
        grid=grid,
        in_specs=[
            pl.BlockSpec((8, _BL), lambda i: (0, i)),
            pl.BlockSpec((_NC, 8, _BL), lambda i: (0, 0, i)),
        ],
        out_specs=pl.BlockSpec((n_cls, _BL), lambda i: (0, i)),
        out_shape=jax.ShapeDtypeStruct((n_cls, n_p), jnp.float32),
    )(f2t, aggt2)


def kernel(x, edge_index, W1a, b1a, W1b, b1b, W2a, b2a, W2b, b2b):
    n, _ = x.shape
    e = edge_index.shape[1]
    h_dim = W1a.shape[1]
    n_cls = W2a.shape[1]

    grp = _NW * _CHUNK * _IDX_BLK
    e_pad = -(-e // grp) * grp
    src = jnp.concatenate([edge_index[0], jnp.zeros((e_pad - e,), jnp.int32)])
    dst = jnp.concatenate([edge_index[1], jnp.full((e_pad - e,), n, jnp.int32)])

    nc16 = e_pad // (_NS * _CHUNK)
    nc32 = e_pad // (_NW * _CHUNK)
    src16_pc = jnp.stack([src, src + n]).reshape(2, _NS, nc16, _CHUNK)
    dst16 = dst.reshape(_NS, nc16, _CHUNK)
    src32 = src.reshape(_NW, nc32, _CHUNK)
    dst32 = dst.reshape(_NW, nc32, _CHUNK)

    n_rows_pad = -(-(n + 1) // (_NS * 8)) * (_NS * 8)
    zrows = jnp.zeros((n_rows_pad // _NS, 8), jnp.float32)

    ht, f1t = _layer1_dense(x.T, W1a.T, b1a, W1b.T, b1b)

    table1 = jnp.concatenate([ht[:8, :n].T, ht[8:, :n].T], axis=0)
    scat_f = _make_edge_scatter(n_rows_pad, nc16, per_core_src=True)
    parts1 = scat_f(table1, src16_pc, dst16, zrows)
    aggt1 = jnp.transpose(parts1, (0, 2, 1))

    wat_p = jnp.zeros((8, h_dim), jnp.float32).at[:n_cls, :].set(W2a.T)
    ba_p = jnp.zeros((8, 1), jnp.float32).at[:n_cls, 0].set(b2a)
    wbt_p = jnp.zeros((8, 8), jnp.float32).at[:n_cls, :n_cls].set(W2b.T)
    bb_p = jnp.zeros((8, 1), jnp.float32).at[:n_cls, 0].set(b2b)
    h2t, f2t = _mid_dense(f1t, aggt1, wat_p, ba_p, wbt_p, bb_p)

    table2 = h2t[:, :n].T
    scat_e = _make_edge_scatter(n_rows_pad, nc32, per_core_src=False)
    parts2 = scat_e(table2, src32, dst32, zrows)
    aggt2 = jnp.transpose(parts2, (0, 2, 1))

    outt = _final_dense(f2t, aggt2, n_cls)
    return outt[:, :n].T


# speedup vs baseline: 1.1959x; 1.1959x over previous
import jax
import jax.numpy as jnp
from jax import lax
from jax.experimental import pallas as pl
from jax.experimental.pallas import tpu as pltpu
from jax.experimental.pallas import tpu_sc as plsc

_NC = 2
_NS = 16
_NW = _NC * _NS
_CHUNK = 128
_IDX_BLK = 8
_BL = 2048


def _mm1_body(xt_ref, wat_ref, ba_ref, wbt_ref, bb_ref, ht_ref, ft_ref):
    ht = jnp.dot(wat_ref[...], xt_ref[...], preferred_element_type=jnp.float32)
    ht = ht + ba_ref[...]
    ht_ref[...] = ht
    ft_ref[...] = (
        jnp.dot(wbt_ref[...], ht, preferred_element_type=jnp.float32)
        + bb_ref[...]
    )


def _layer1_dense(xt, wat, ba, wbt, bb):
    f_in, n = xt.shape
    h_dim = wat.shape[0]
    grid = (-(-n // _BL),)
    n_p = grid[0] * _BL
    return pl.pallas_call(
        _mm1_body,
        grid=grid,
        in_specs=[
            pl.BlockSpec((f_in, _BL), lambda i: (0, i)),
            pl.BlockSpec((h_dim, f_in), lambda i: (0, 0)),
            pl.BlockSpec((h_dim, 1), lambda i: (0, 0)),
            pl.BlockSpec((h_dim, h_dim), lambda i: (0, 0)),
            pl.BlockSpec((h_dim, 1), lambda i: (0, 0)),
        ],
        out_specs=[
            pl.BlockSpec((h_dim, _BL), lambda i: (0, i)),
            pl.BlockSpec((h_dim, _BL), lambda i: (0, i)),
        ],
        out_shape=[
            jax.ShapeDtypeStruct((h_dim, n_p), jnp.float32),
            jax.ShapeDtypeStruct((h_dim, n_p), jnp.float32),
        ],
    )(xt, wat, ba.reshape(-1, 1), wbt, bb.reshape(-1, 1))


def _make_edge_scatter(n_rows_pad, n_chunks, per_core_src):
    rows_per_tile = n_rows_pad // _NS
    n_outer = n_chunks // _IDX_BLK
    mesh = plsc.VectorSubcoreMesh(core_axis_name="c", subcore_axis_name="s")

    def body(table_hbm, src_hbm, dst_hbm, z_hbm, out_hbm,
             srcb, dstb, grow, acc, gsem, ssem):
        c = lax.axis_index("c")
        s = lax.axis_index("s")

        pltpu.sync_copy(z_hbm, acc.at[pl.ds(s * rows_per_tile, rows_per_tile)])
        plsc.subcore_barrier()

        def outer(ob, carry):
            base = ob * _IDX_BLK
            if per_core_src:
                pltpu.sync_copy(src_hbm.at[c, s, pl.ds(base, _IDX_BLK)], srcb)
                pltpu.sync_copy(dst_hbm.at[s, pl.ds(base, _IDX_BLK)], dstb)
            else:
                wid = s * _NC + c
                pltpu.sync_copy(src_hbm.at[wid, pl.ds(base, _IDX_BLK)], srcb)
                pltpu.sync_copy(dst_hbm.at[wid, pl.ds(base, _IDX_BLK)], dstb)
            gathers = [
                pltpu.async_copy(table_hbm.at[srcb.at[j]], grow.at[j], gsem)
                for j in range(_IDX_BLK)
            ]
            for g in gathers:
                g.wait()
            scats = [
                pltpu.async_copy(grow.at[j], acc.at[dstb.at[j]], ssem, add=True)
                for j in range(_IDX_BLK)
            ]
            for sc in scats:
                sc.wait()
            return carry

        lax.fori_loop(0, n_outer, outer, 0)
        plsc.subcore_barrier()
        pltpu.sync_copy(
            acc.at[pl.ds(s * rows_per_tile, rows_per_tile)],
            out_hbm.at[c, pl.ds(s * rows_per_tile, rows_per_tile)],
        )

    return pl.kernel(
        body,
        out_type=jax.ShapeDtypeStruct((_NC, n_rows_pad, 8), jnp.float32),
        mesh=mesh,
        compiler_params=pltpu.CompilerParams(use_tc_tiling_on_sc=False),
        scratch_types=[
            pltpu.VMEM((_IDX_BLK, _CHUNK), jnp.int32),
            pltpu.VMEM((_IDX_BLK, _CHUNK), jnp.int32),
            pltpu.VMEM((_IDX_BLK, _CHUNK, 8), jnp.float32),
            pltpu.VMEM_SHARED((n_rows_pad, 8), jnp.float32),
            pltpu.SemaphoreType.DMA,
            pltpu.SemaphoreType.DMA,
        ],
    )


def _mid_body(f_ref, p_ref, wat_ref, ba_ref, wbt_ref, bb_ref, h2_ref, f2_ref):
    agg = jnp.concatenate([p_ref[0], p_ref[1]], axis=0)
    out1 = jnp.maximum(f_ref[...] + agg, 0.0)
    h2 = jnp.dot(wat_ref[...], out1, preferred_element_type=jnp.float32)
    h2 = h2 + ba_ref[...]
    h2_ref[...] = h2
    f2_ref[...] = (
        jnp.dot(wbt_ref[...], h2, preferred_element_type=jnp.float32)
        + bb_ref[...]
    )


def _mid_dense(f1t, aggt, wat_p, ba_p, wbt_p, bb_p):
    h_dim, n_p = f1t.shape
    grid = (n_p // _BL,)
    return pl.pallas_call(
        _mid_body,
        grid=grid,
        in_specs=[
            pl.BlockSpec((h_dim, _BL), lambda i: (0, i)),
            pl.BlockSpec((_NC, 8, _BL), lambda i: (0, 0, i)),
            pl.BlockSpec((8, h_dim), lambda i: (0, 0)),
            pl.BlockSpec((8, 1), lambda i: (0, 0)),
            pl.BlockSpec((8, 8), lambda i: (0, 0)),
            pl.BlockSpec((8, 1), lambda i: (0, 0)),
        ],
        out_specs=[
            pl.BlockSpec((8, _BL), lambda i: (0, i)),
            pl.BlockSpec((8, _BL), lambda i: (0, i)),
        ],
        out_shape=[
            jax.ShapeDtypeStruct((8, n_p), jnp.float32),
            jax.ShapeDtypeStruct((8, n_p), jnp.float32),
        ],
    )(f1t, aggt, wat_p, ba_p, wbt_p, bb_p)


def _make_final_body(n_cls):
    def _final_body(f2_ref, p_ref, o_ref):
        z = f2_ref[...] + p_ref[0] + p_ref[1]
        row = lax.broadcasted_iota(jnp.int32, z.shape, 0)
        z = jnp.where(row < n_cls, z, -jnp.inf)
        m = jnp.max(z, axis=0, keepdims=True)
        zs = z - m
        lse = jnp.log(jnp.sum(jnp.exp(zs), axis=0, keepdims=True))
        o_ref[...] = (zs - lse)[:n_cls, :]
    return _final_body


def _final_dense(f2t, aggt2, n_cls):
    _, n_p = f2t.shape
    grid = (n_p // _BL,)
    return pl.pallas_call(
        _make_final_body(n_cls),
        grid=grid,
        in_specs=[
            pl.BlockSpec((8, _BL), lambda i: (0, i)),
            pl.BlockSpec((_NC, 8, _BL), lambda i: (0, 0, i)),
        ],
        out_specs=pl.BlockSpec((n_cls, _BL), lambda i: (0, i)),
        out_shape=jax.ShapeDtypeStruct((n_cls, n_p), jnp.float32),
    )(f2t, aggt2)


def kernel(x, edge_index, W1a, b1a, W1b, b1b, W2a, b2a, W2b, b2b):
    n, _ = x.shape
    e = edge_index.shape[1]
    h_dim = W1a.shape[1]
    n_cls = W2a.shape[1]

    grp = _NW * _CHUNK * _IDX_BLK
    e_pad = -(-e // grp) * grp
    src = jnp.concatenate([edge_index[0], jnp.zeros((e_pad - e,), jnp.int32)])
    dst = jnp.concatenate([edge_index[1], jnp.full((e_pad - e,), n, jnp.int32)])

    nc16 = e_pad // (_NS * _CHUNK)
    nc32 = e_pad // (_NW * _CHUNK)
    src16_pc = jnp.stack([src, src + n]).reshape(2, _NS, nc16, _CHUNK)
    dst16 = dst.reshape(_NS, nc16, _CHUNK)
    src32 = src.reshape(_NW, nc32, _CHUNK)
    dst32 = dst.reshape(_NW, nc32, _CHUNK)

    n_rows_pad = -(-(n + 1) // (_NS * 8)) * (_NS * 8)
    zrows = jnp.zeros((n_rows_pad // _NS, 8), jnp.float32)

    ht, f1t = _layer1_dense(x.T, W1a.T, b1a, W1b.T, b1b)

    table1 = jnp.concatenate([ht[:8, :n].T, ht[8:, :n].T], axis=0)
    scat_f = _make_edge_scatter(n_rows_pad, nc16, per_core_src=True)
    parts1 = scat_f(table1, src16_pc, dst16, zrows)
    aggt1 = jnp.transpose(parts1, (0, 2, 1))

    wat_p = jnp.zeros((8, h_dim), jnp.float32).at[:n_cls, :].set(W2a.T)
    ba_p = jnp.zeros((8, 1), jnp.float32).at[:n_cls, 0].set(b2a)
    wbt_p = jnp.zeros((8, 8), jnp.float32).at[:n_cls, :n_cls].set(W2b.T)
    bb_p = jnp.zeros((8, 1), jnp.float32).at[:n_cls, 0].set(b2b)
    h2t, f2t = _mid_dense(f1t, aggt1, wat_p, ba_p, wbt_p, bb_p)

    table2 = h2t[:, :n].T
    scat_e = _make_edge_scatter(n_rows_pad, nc32, per_core_src=False)
    parts2 = scat_e(table2, src32, dst32, zrows)
    aggt2 = jnp.transpose(parts2, (0, 2, 1))

    outt = _final_dense(f2t, aggt2, n_cls)
    return outt[:, :n].T


# speedup vs baseline: 1.3841x; 1.1573x over previous
import jax
import jax.numpy as jnp
from jax import lax
from jax.experimental import pallas as pl
from jax.experimental.pallas import tpu as pltpu
from jax.experimental.pallas import tpu_sc as plsc

_NC = 2
_NS = 16
_NW = _NC * _NS
_CHUNK = 128
_IDX_BLK = 16
_BL = 4096


def _mm1_body(xt_ref, wat_ref, ba_ref, wbt_ref, bb_ref, ht_ref, ft_ref):
    ht = jnp.dot(wat_ref[...], xt_ref[...], preferred_element_type=jnp.float32)
    ht = ht + ba_ref[...]
    ht_ref[...] = ht
    ft_ref[...] = (
        jnp.dot(wbt_ref[...], ht, preferred_element_type=jnp.float32)
        + bb_ref[...]
    )


def _layer1_dense(xt, wat, ba, wbt, bb):
    f_in, n = xt.shape
    h_dim = wat.shape[0]
    grid = (-(-n // _BL),)
    n_p = grid[0] * _BL
    return pl.pallas_call(
        _mm1_body,
        grid=grid,
        in_specs=[
            pl.BlockSpec((f_in, _BL), lambda i: (0, i)),
            pl.BlockSpec((h_dim, f_in), lambda i: (0, 0)),
            pl.BlockSpec((h_dim, 1), lambda i: (0, 0)),
            pl.BlockSpec((h_dim, h_dim), lambda i: (0, 0)),
            pl.BlockSpec((h_dim, 1), lambda i: (0, 0)),
        ],
        out_specs=[
            pl.BlockSpec((h_dim, _BL), lambda i: (0, i)),
            pl.BlockSpec((h_dim, _BL), lambda i: (0, i)),
        ],
        out_shape=[
            jax.ShapeDtypeStruct((h_dim, n_p), jnp.float32),
            jax.ShapeDtypeStruct((h_dim, n_p), jnp.float32),
        ],
    )(xt, wat, ba.reshape(-1, 1), wbt, bb.reshape(-1, 1))


def _make_edge_scatter(n_rows_pad, n_chunks, per_core_src):
    rows_per_tile = n_rows_pad // _NS
    n_outer = n_chunks // _IDX_BLK
    mesh = plsc.VectorSubcoreMesh(core_axis_name="c", subcore_axis_name="s")

    def body(table_hbm, src_hbm, dst_hbm, z_hbm, out_hbm,
             srcb, dstb, grow, acc, gsem, ssem):
        c = lax.axis_index("c")
        s = lax.axis_index("s")

        pltpu.sync_copy(z_hbm, acc.at[pl.ds(s * rows_per_tile, rows_per_tile)])
        plsc.subcore_barrier()

        def outer(ob, carry):
            base = ob * _IDX_BLK
            if per_core_src:
                pltpu.sync_copy(src_hbm.at[c, s, pl.ds(base, _IDX_BLK)], srcb)
                pltpu.sync_copy(dst_hbm.at[s, pl.ds(base, _IDX_BLK)], dstb)
            else:
                wid = s * _NC + c
                pltpu.sync_copy(src_hbm.at[wid, pl.ds(base, _IDX_BLK)], srcb)
                pltpu.sync_copy(dst_hbm.at[wid, pl.ds(base, _IDX_BLK)], dstb)
            gathers = [
                pltpu.async_copy(table_hbm.at[srcb.at[j]], grow.at[j], gsem)
                for j in range(_IDX_BLK)
            ]
            for g in gathers:
                g.wait()
            scats = [
                pltpu.async_copy(grow.at[j], acc.at[dstb.at[j]], ssem, add=True)
                for j in range(_IDX_BLK)
            ]
            for sc in scats:
                sc.wait()
            return carry

        lax.fori_loop(0, n_outer, outer, 0)
        plsc.subcore_barrier()
        pltpu.sync_copy(
            acc.at[pl.ds(s * rows_per_tile, rows_per_tile)],
            out_hbm.at[c, pl.ds(s * rows_per_tile, rows_per_tile)],
        )

    return pl.kernel(
        body,
        out_type=jax.ShapeDtypeStruct((_NC, n_rows_pad, 8), jnp.float32),
        mesh=mesh,
        compiler_params=pltpu.CompilerParams(use_tc_tiling_on_sc=False),
        scratch_types=[
            pltpu.VMEM((_IDX_BLK, _CHUNK), jnp.int32),
            pltpu.VMEM((_IDX_BLK, _CHUNK), jnp.int32),
            pltpu.VMEM((_IDX_BLK, _CHUNK, 8), jnp.float32),
            pltpu.VMEM_SHARED((n_rows_pad, 8), jnp.float32),
            pltpu.SemaphoreType.DMA,
            pltpu.SemaphoreType.DMA,
        ],
    )


def _mid_body(f_ref, p_ref, wat_ref, ba_ref, wbt_ref, bb_ref, h2_ref, f2_ref):
    agg = jnp.concatenate([p_ref[0], p_ref[1]], axis=0)
    out1 = jnp.maximum(f_ref[...] + agg, 0.0)
    h2 = jnp.dot(wat_ref[...], out1, preferred_element_type=jnp.float32)
    h2 = h2 + ba_ref[...]
    h2_ref[...] = h2
    f2_ref[...] = (
        jnp.dot(wbt_ref[...], h2, preferred_element_type=jnp.float32)
        + bb_ref[...]
    )


def _mid_dense(f1t, aggt, wat_p, ba_p, wbt_p, bb_p):
    h_dim, n_p = f1t.shape
    grid = (n_p // _BL,)
    return pl.pallas_call(
        _mid_body,
        grid=grid,
        in_specs=[
            pl.BlockSpec((h_dim, _BL), lambda i: (0, i)),
            pl.BlockSpec((_NC, 8, _BL), lambda i: (0, 0, i)),
            pl.BlockSpec((8, h_dim), lambda i: (0, 0)),
            pl.BlockSpec((8, 1), lambda i: (0, 0)),
            pl.BlockSpec((8, 8), lambda i: (0, 0)),
            pl.BlockSpec((8, 1), lambda i: (0, 0)),
        ],
        out_specs=[
            pl.BlockSpec((8, _BL), lambda i: (0, i)),
            pl.BlockSpec((8, _BL), lambda i: (0, i)),
        ],
        out_shape=[
            jax.ShapeDtypeStruct((8, n_p), jnp.float32),
            jax.ShapeDtypeStruct((8, n_p), jnp.float32),
        ],
    )(f1t, aggt, wat_p, ba_p, wbt_p, bb_p)


def _make_final_body(n_cls):
    def _final_body(f2_ref, p_ref, o_ref):
        z = f2_ref[...] + p_ref[0] + p_ref[1]
        row = lax.broadcasted_iota(jnp.int32, z.shape, 0)
        z = jnp.where(row < n_cls, z, -jnp.inf)
        m = jnp.max(z, axis=0, keepdims=True)
        zs = z - m
        lse = jnp.log(jnp.sum(jnp.exp(zs), axis=0, keepdims=True))
        o_ref[...] = (zs - lse)[:n_cls, :]
    return _final_body


def _final_dense(f2t, aggt2, n_cls):
    _, n_p = f2t.shape
    grid = (n_p // _BL,)
    return pl.pallas_call(
        _make_final_body(n_cls),
        grid=grid,
        in_specs=[
            pl.BlockSpec((8, _BL), lambda i: (0, i)),
            pl.BlockSpec((_NC, 8, _BL), lambda i: (0, 0, i)),
        ],
        out_specs=pl.BlockSpec((n_cls, _BL), lambda i: (0, i)),
        out_shape=jax.ShapeDtypeStruct((n_cls, n_p), jnp.float32),
    )(f2t, aggt2)


def kernel(x, edge_index, W1a, b1a, W1b, b1b, W2a, b2a, W2b, b2b):
    n, _ = x.shape
    e = edge_index.shape[1]
    h_dim = W1a.shape[1]
    n_cls = W2a.shape[1]

    grp = _NW * _CHUNK * _IDX_BLK
    e_pad = -(-e // grp) * grp
    src = jnp.concatenate([edge_index[0], jnp.zeros((e_pad - e,), jnp.int32)])
    dst = jnp.concatenate([edge_index[1], jnp.full((e_pad - e,), n, jnp.int32)])

    nc16 = e_pad // (_NS * _CHUNK)
    nc32 = e_pad // (_NW * _CHUNK)
    src16_pc = jnp.stack([src, src + n]).reshape(2, _NS, nc16, _CHUNK)
    dst16 = dst.reshape(_NS, nc16, _CHUNK)
    src32 = src.reshape(_NW, nc32, _CHUNK)
    dst32 = dst.reshape(_NW, nc32, _CHUNK)

    n_rows_pad = -(-(n + 1) // (_NS * 8)) * (_NS * 8)
    zrows = jnp.zeros((n_rows_pad // _NS, 8), jnp.float32)

    ht, f1t = _layer1_dense(x.T, W1a.T, b1a, W1b.T, b1b)

    table1 = jnp.concatenate([ht[:8, :n].T, ht[8:, :n].T], axis=0)
    scat_f = _make_edge_scatter(n_rows_pad, nc16, per_core_src=True)
    parts1 = scat_f(table1, src16_pc, dst16, zrows)
    aggt1 = jnp.transpose(parts1, (0, 2, 1))

    wat_p = jnp.zeros((8, h_dim), jnp.float32).at[:n_cls, :].set(W2a.T)
    ba_p = jnp.zeros((8, 1), jnp.float32).at[:n_cls, 0].set(b2a)
    wbt_p = jnp.zeros((8, 8), jnp.float32).at[:n_cls, :n_cls].set(W2b.T)
    bb_p = jnp.zeros((8, 1), jnp.float32).at[:n_cls, 0].set(b2b)
    h2t, f2t = _mid_dense(f1t, aggt1, wat_p, ba_p, wbt_p, bb_p)

    table2 = h2t[:, :n].T
    scat_e = _make_edge_scatter(n_rows_pad, nc32, per_core_src=False)
    parts2 = scat_e(table2, src32, dst32, zrows)
    aggt2 = jnp.transpose(parts2, (0, 2, 1))

    outt = _final_dense(f2t, aggt2, n_cls)
    return outt[:, :n].T
